# baseline (speedup 1.0000x reference)
import functools

import jax
import jax.numpy as jnp
from jax import lax
from jax.experimental import pallas as pl
from jax.experimental.pallas import tpu as pltpu
from jax.experimental.pallas import tpu_sc as plsc

D = 768
B, S = 4, 8192
NTOK = B * S
NC, NS, L = 2, 16, 16
NW = NC * NS
TOK_PER = NTOK // NW
CHUNK = 16
NCHUNK = TOK_PER // CHUNK
NROWS = 24 + 7 + 31 + 12
OFF_H, OFF_W, OFF_D, OFF_M = 0, 24, 31, 62


def _make_sc_kernel():
    mesh = plsc.VectorSubcoreMesh(core_axis_name="c", subcore_axis_name="s")

    @functools.partial(
        pl.kernel,
        mesh=mesh,
        out_type=jax.ShapeDtypeStruct((NTOK, D), jnp.float32),
        scratch_types=[
            pltpu.VMEM((4, TOK_PER), jnp.int32),
            pltpu.VMEM((NROWS, D), jnp.float32),
            pltpu.VMEM((2, CHUNK, D), jnp.float32),
            pltpu.SemaphoreType.DMA,
            pltpu.SemaphoreType.DMA,
        ],
    )
    def body(xt_hbm, tab_hbm, out_hbm, idx_v, tab_v, stage, sem0, sem1):
        wid = lax.axis_index("s") * NC + lax.axis_index("c")
        base = wid * TOK_PER
        pltpu.sync_copy(tab_hbm, tab_v)
        for c in range(4):
            pltpu.sync_copy(xt_hbm.at[c, pl.ds(base, TOK_PER)], idx_v.at[c])

        sems = (sem0, sem1)

        def compute_chunk(ci, b):
            tok = pl.ds(ci * CHUNK, CHUNK)
            rm = idx_v[0, tok] + (OFF_M - 1)
            rd = idx_v[1, tok] + (OFF_D - 1)
            rw = idx_v[2, tok] + OFF_W
            rh = idx_v[3, tok] + OFF_H

            for tl in range(CHUNK):

                def dim_body(j, carry3, tl=tl):
                    col = pl.ds(j * L, L)
                    stage[b, tl, col] = (
                        tab_v[rh[tl], col]
                        + tab_v[rw[tl], col]
                        + tab_v[rd[tl], col]
                        + tab_v[rm[tl], col]
                    )
                    return carry3

                lax.fori_loop(0, D // L, dim_body, 0, unroll=8)

        def pair_body(pi, carry):
            for b in range(2):
                ci = pi * 2 + b

                @pl.when(pi > 0)
                def _(b=b):
                    pltpu.make_async_copy(
                        stage.at[b], out_hbm.at[pl.ds(base, CHUNK)], sems[b]
                    ).wait()

                compute_chunk(ci, b)
                pltpu.make_async_copy(
                    stage.at[b],
                    out_hbm.at[pl.ds(base + ci * CHUNK, CHUNK)],
                    sems[b],
                ).start()
            return carry

        lax.fori_loop(0, NCHUNK // 2, pair_body, 0)
        for b in range(2):
            pltpu.make_async_copy(
                stage.at[b], out_hbm.at[pl.ds(base, CHUNK)], sems[b]
            ).wait()

    return body


_sc_lookup = _make_sc_kernel()


def kernel(x, hour_w, weekday_w, day_w, month_w):
    xt = x.astype(jnp.int32).reshape(NTOK, 4).T
    table = jnp.concatenate([hour_w, weekday_w, day_w, month_w], axis=0)
    out = _sc_lookup(xt, table)
    return out.reshape(B, S, D)


# speedup vs baseline: 2.3263x; 2.3263x over previous
import functools

import jax
import jax.numpy as jnp
from jax import lax
from jax.experimental import pallas as pl
from jax.experimental.pallas import tpu as pltpu
from jax.experimental.pallas import tpu_sc as plsc

D = 768
B, S = 4, 8192
NTOK = B * S
NC, NS, L = 2, 16, 16
NW = NC * NS
TOK_PER = NTOK // NW
CHUNK = 16
NCHUNK = TOK_PER // CHUNK
NROWS = 24 + 7 + 31 + 12
OFF_H, OFF_W, OFF_D, OFF_M = 0, 24, 31, 62


def _make_sc_kernel():
    mesh = plsc.VectorSubcoreMesh(core_axis_name="c", subcore_axis_name="s")

    @functools.partial(
        pl.kernel,
        mesh=mesh,
        out_type=jax.ShapeDtypeStruct((NTOK, D), jnp.float32),
        scratch_types=[
            pltpu.VMEM((4, TOK_PER), jnp.int32),
            pltpu.VMEM((NROWS, D), jnp.float32),
            pltpu.VMEM((2, CHUNK, D), jnp.float32),
            pltpu.SemaphoreType.DMA,
            pltpu.SemaphoreType.DMA,
        ],
    )
    def body(xt_hbm, tab_hbm, out_hbm, idx_v, tab_v, stage, sem0, sem1):
        wid = lax.axis_index("s") * NC + lax.axis_index("c")
        base = wid * TOK_PER
        pltpu.sync_copy(tab_hbm, tab_v)
        for c in range(4):
            pltpu.sync_copy(xt_hbm.at[c, pl.ds(base, TOK_PER)], idx_v.at[c])

        sems = (sem0, sem1)

        def compute_chunk(ci, b):
            tok = pl.ds(ci * CHUNK, CHUNK)
            rm = idx_v[0, tok] + (OFF_M - 1)
            rd = idx_v[1, tok] + (OFF_D - 1)
            rw = idx_v[2, tok] + OFF_W
            rh = idx_v[3, tok] + OFF_H

            for tl in range(CHUNK):

                @plsc.parallel_loop(0, D // L, unroll=8)
                def dim_body(j, tl=tl):
                    col = pl.ds(j * L, L)
                    stage[b, tl, col] = (
                        tab_v[rh[tl], col]
                        + tab_v[rw[tl], col]
                        + tab_v[rd[tl], col]
                        + tab_v[rm[tl], col]
                    )

        def pair_body(pi, carry):
            for b in range(2):
                ci = pi * 2 + b

                @pl.when(pi > 0)
                def _(b=b):
                    pltpu.make_async_copy(
                        stage.at[b], out_hbm.at[pl.ds(base, CHUNK)], sems[b]
                    ).wait()

                compute_chunk(ci, b)
                pltpu.make_async_copy(
                    stage.at[b],
                    out_hbm.at[pl.ds(base + ci * CHUNK, CHUNK)],
                    sems[b],
                ).start()
            return carry

        lax.fori_loop(0, NCHUNK // 2, pair_body, 0)
        for b in range(2):
            pltpu.make_async_copy(
                stage.at[b], out_hbm.at[pl.ds(base, CHUNK)], sems[b]
            ).wait()

    return body


_sc_lookup = _make_sc_kernel()


def kernel(x, hour_w, weekday_w, day_w, month_w):
    xt = x.astype(jnp.int32).reshape(NTOK, 4).T
    table = jnp.concatenate([hour_w, weekday_w, day_w, month_w], axis=0)
    out = _sc_lookup(xt, table)
    return out.reshape(B, S, D)
